# baseline (speedup 1.0000x reference)
import jax
import jax.numpy as jnp
from jax.experimental import pallas as pl
from jax.experimental.pallas import tpu as pltpu

OBS_DIM = 512
ACT_DIM = 32
HID = 1024
LAT = 64
K = 1024
RB = 2048


def _body(obs_ref, act_ref, We1_ref, be1_ref, We2_ref, be2_ref, We3_ref,
          be3_ref, cb_ref, c2_ref, Wd1q_ref, Wd1a_ref, bd1_ref, Wd2_ref,
          bd2_ref, Wd3_ref, bd3_ref,
          recon_ref, tok_ref, qst_ref, lat_ref, vqp_ref, rcp_ref):
    f32 = jnp.float32
    x = obs_ref[...]
    a = act_ref[...]
    enc_in = jnp.concatenate((x, a), axis=1)
    h = jnp.maximum(jnp.dot(enc_in, We1_ref[...], preferred_element_type=f32)
                    + be1_ref[...], 0.0)
    h = jnp.maximum(jnp.dot(h, We2_ref[...], preferred_element_type=f32)
                    + be2_ref[...], 0.0)
    lat = jnp.dot(h, We3_ref[...], preferred_element_type=f32) + be3_ref[...]

    x2 = jnp.sum(lat * lat, axis=1, keepdims=True)
    xc = jax.lax.dot_general(lat, cb_ref[...], (((1,), (1,)), ((), ())),
                             preferred_element_type=f32)
    d2 = x2 - 2.0 * xc + c2_ref[...]
    col = jax.lax.broadcasted_iota(jnp.int32, (RB, K), 1)
    minv = jnp.min(d2, axis=1, keepdims=True)
    idx = jnp.min(jnp.where(d2 == minv, col, K), axis=1)

    onehot = (col == idx[:, None]).astype(f32)
    quant = jnp.dot(onehot, cb_ref[...], preferred_element_type=f32)
    qst = lat + (quant - lat)

    bf16 = jnp.bfloat16
    hd = jnp.maximum(
        jnp.dot(qst.astype(bf16), Wd1q_ref[...], preferred_element_type=f32)
        + jnp.dot(a.astype(bf16), Wd1a_ref[...], preferred_element_type=f32)
        + bd1_ref[...], 0.0)
    hd = jnp.maximum(
        jnp.dot(hd.astype(bf16), Wd2_ref[...], preferred_element_type=f32)
        + bd2_ref[...], 0.0)
    recon = jnp.dot(hd.astype(bf16), Wd3_ref[...],
                    preferred_element_type=f32) + bd3_ref[...]

    recon_ref[...] = recon
    tok_ref[...] = idx[:, None]
    qst_ref[...] = qst
    lat_ref[...] = lat
    vqp_ref[...] = jnp.broadcast_to(jnp.sum((lat - quant) ** 2), (1, 1, 128))
    rcp_ref[...] = jnp.broadcast_to(jnp.sum((recon - x) ** 2), (1, 1, 128))


def kernel(obs, actions, We1, be1, We2, be2, We3, be3, codebook,
           Wd1, bd1, Wd2, bd2, Wd3, bd3):
    b, s = obs.shape[0], obs.shape[1]
    n = b * s
    grid = n // RB
    f32 = jnp.float32

    obs_f = obs.reshape(n, OBS_DIM)
    act_f = actions.reshape(n, ACT_DIM)
    c2 = jnp.sum(codebook * codebook, axis=1)[None, :]

    row = lambda rb, cols: pl.BlockSpec((rb, cols), lambda i: (i, 0))
    rep = lambda shape: pl.BlockSpec(shape, lambda i: (0,) * len(shape))

    out_shapes = (
        jax.ShapeDtypeStruct((n, OBS_DIM), f32),
        jax.ShapeDtypeStruct((n, 1), jnp.int32),
        jax.ShapeDtypeStruct((n, LAT), f32),
        jax.ShapeDtypeStruct((n, LAT), f32),
        jax.ShapeDtypeStruct((grid, 1, 128), f32),
        jax.ShapeDtypeStruct((grid, 1, 128), f32),
    )
    out_specs = (
        row(RB, OBS_DIM),
        row(RB, 1),
        row(RB, LAT),
        row(RB, LAT),
        pl.BlockSpec((1, 1, 128), lambda i: (i, 0, 0)),
        pl.BlockSpec((1, 1, 128), lambda i: (i, 0, 0)),
    )
    in_specs = (
        row(RB, OBS_DIM),
        row(RB, ACT_DIM),
        rep((OBS_DIM + ACT_DIM, HID)),
        rep((1, HID)),
        rep((HID, HID)),
        rep((1, HID)),
        rep((HID, LAT)),
        rep((1, LAT)),
        rep((K, LAT)),
        rep((1, K)),
        rep((LAT, HID)),
        rep((ACT_DIM, HID)),
        rep((1, HID)),
        rep((HID, HID)),
        rep((1, HID)),
        rep((HID, OBS_DIM)),
        rep((1, OBS_DIM)),
    )

    recon_f, tok, qst_f, lat_f, vqp, rcp = pl.pallas_call(
        _body,
        grid=(grid,),
        in_specs=list(in_specs),
        out_specs=list(out_specs),
        out_shape=out_shapes,
        compiler_params=pltpu.CompilerParams(
            vmem_limit_bytes=100 * 1024 * 1024,
        ),
    )(obs_f, act_f, We1, be1[None, :], We2, be2[None, :], We3, be3[None, :],
      codebook, c2, Wd1[:LAT].astype(jnp.bfloat16),
      Wd1[LAT:].astype(jnp.bfloat16), bd1[None, :],
      Wd2.astype(jnp.bfloat16), bd2[None, :],
      Wd3.astype(jnp.bfloat16), bd3[None, :])

    reconstructed_obs = recon_f.reshape(b, s, OBS_DIM)
    tokens = tok.reshape(b, s)
    quantized_st = qst_f.reshape(b, s, LAT)
    latents = lat_f.reshape(b, s, LAT)

    ssq_vq = jnp.sum(vqp[:, 0, 0])
    ssq_rc = jnp.sum(rcp[:, 0, 0])
    msd = ssq_vq / jnp.float32(n * LAT)
    commitment_loss = msd * 0.25
    codebook_loss = msd
    recon_loss = ssq_rc / jnp.float32(n * OBS_DIM)
    total_quantizer_loss = commitment_loss + codebook_loss
    total_tokenizer_loss = recon_loss + total_quantizer_loss
    return (reconstructed_obs, tokens, quantized_st, latents, recon_loss,
            commitment_loss, codebook_loss, total_quantizer_loss,
            total_tokenizer_loss)


# speedup vs baseline: 1.0267x; 1.0267x over previous
import jax
import jax.numpy as jnp
from jax.experimental import pallas as pl
from jax.experimental.pallas import tpu as pltpu

OBS_DIM = 512
ACT_DIM = 32
HID = 1024
LAT = 64
K = 1024
RB = 2048


def _body(obs_ref, act_ref, We1_ref, be1_ref, We2_ref, be2_ref, We3_ref,
          be3_ref, cb_ref, c2_ref, Wd1_ref, bd1_ref, Wd2_ref,
          bd2_ref, Wd3_ref, bd3_ref,
          recon_ref, tok_ref, qst_ref, lat_ref, vqp_ref, rcp_ref):
    f32 = jnp.float32
    x = obs_ref[...]
    a = act_ref[...]
    enc_in = jnp.concatenate((x, a), axis=1)
    h = jnp.maximum(jnp.dot(enc_in, We1_ref[...], preferred_element_type=f32)
                    + be1_ref[...], 0.0)
    h = jnp.maximum(jnp.dot(h, We2_ref[...], preferred_element_type=f32)
                    + be2_ref[...], 0.0)
    lat = jnp.dot(h, We3_ref[...], preferred_element_type=f32) + be3_ref[...]

    x2 = jnp.sum(lat * lat, axis=1, keepdims=True)
    xc = jax.lax.dot_general(lat, cb_ref[...], (((1,), (1,)), ((), ())),
                             preferred_element_type=f32)
    d2 = x2 - 2.0 * xc + c2_ref[...]
    col = jax.lax.broadcasted_iota(jnp.int32, (RB, K), 1)
    minv = jnp.min(d2, axis=1, keepdims=True)
    idx = jnp.min(jnp.where(d2 == minv, col, K), axis=1)

    onehot = (col == idx[:, None]).astype(f32)
    quant = jnp.dot(onehot, cb_ref[...], preferred_element_type=f32)
    qst = lat + (quant - lat)

    hd = jnp.maximum(
        jnp.dot(qst, Wd1_ref[...][:LAT], preferred_element_type=f32)
        + jnp.dot(a, Wd1_ref[...][LAT:], preferred_element_type=f32)
        + bd1_ref[...], 0.0)
    hd = jnp.maximum(
        jnp.dot(hd, Wd2_ref[...], preferred_element_type=f32)
        + bd2_ref[...], 0.0)
    recon = jnp.dot(hd, Wd3_ref[...], preferred_element_type=f32) + bd3_ref[...]

    recon_ref[...] = recon
    tok_ref[...] = idx[:, None]
    qst_ref[...] = qst
    lat_ref[...] = lat
    vqp_ref[...] = jnp.broadcast_to(jnp.sum((lat - quant) ** 2), (1, 1, 128))
    rcp_ref[...] = jnp.broadcast_to(jnp.sum((recon - x) ** 2), (1, 1, 128))


def kernel(obs, actions, We1, be1, We2, be2, We3, be3, codebook,
           Wd1, bd1, Wd2, bd2, Wd3, bd3):
    b, s = obs.shape[0], obs.shape[1]
    n = b * s
    grid = n // RB
    f32 = jnp.float32

    obs_f = obs.reshape(n, OBS_DIM)
    act_f = actions.reshape(n, ACT_DIM)
    c2 = jnp.sum(codebook * codebook, axis=1)[None, :]

    row = lambda rb, cols: pl.BlockSpec((rb, cols), lambda i: (i, 0))
    rep = lambda shape: pl.BlockSpec(shape, lambda i: (0,) * len(shape))

    out_shapes = (
        jax.ShapeDtypeStruct((n, OBS_DIM), f32),
        jax.ShapeDtypeStruct((n, 1), jnp.int32),
        jax.ShapeDtypeStruct((n, LAT), f32),
        jax.ShapeDtypeStruct((n, LAT), f32),
        jax.ShapeDtypeStruct((grid, 1, 128), f32),
        jax.ShapeDtypeStruct((grid, 1, 128), f32),
    )
    out_specs = (
        row(RB, OBS_DIM),
        row(RB, 1),
        row(RB, LAT),
        row(RB, LAT),
        pl.BlockSpec((1, 1, 128), lambda i: (i, 0, 0)),
        pl.BlockSpec((1, 1, 128), lambda i: (i, 0, 0)),
    )
    in_specs = (
        row(RB, OBS_DIM),
        row(RB, ACT_DIM),
        rep((OBS_DIM + ACT_DIM, HID)),
        rep((1, HID)),
        rep((HID, HID)),
        rep((1, HID)),
        rep((HID, LAT)),
        rep((1, LAT)),
        rep((K, LAT)),
        rep((1, K)),
        rep((LAT + ACT_DIM, HID)),
        rep((1, HID)),
        rep((HID, HID)),
        rep((1, HID)),
        rep((HID, OBS_DIM)),
        rep((1, OBS_DIM)),
    )

    recon_f, tok, qst_f, lat_f, vqp, rcp = pl.pallas_call(
        _body,
        grid=(grid,),
        in_specs=list(in_specs),
        out_specs=list(out_specs),
        out_shape=out_shapes,
        compiler_params=pltpu.CompilerParams(
            vmem_limit_bytes=100 * 1024 * 1024,
        ),
    )(obs_f, act_f, We1, be1[None, :], We2, be2[None, :], We3, be3[None, :],
      codebook, c2, Wd1, bd1[None, :], Wd2, bd2[None, :], Wd3, bd3[None, :])

    reconstructed_obs = recon_f.reshape(b, s, OBS_DIM)
    tokens = tok.reshape(b, s)
    quantized_st = qst_f.reshape(b, s, LAT)
    latents = lat_f.reshape(b, s, LAT)

    ssq_vq = jnp.sum(vqp[:, 0, 0])
    ssq_rc = jnp.sum(rcp[:, 0, 0])
    msd = ssq_vq / jnp.float32(n * LAT)
    commitment_loss = msd * 0.25
    codebook_loss = msd
    recon_loss = ssq_rc / jnp.float32(n * OBS_DIM)
    total_quantizer_loss = commitment_loss + codebook_loss
    total_tokenizer_loss = recon_loss + total_quantizer_loss
    return (reconstructed_obs, tokens, quantized_st, latents, recon_loss,
            commitment_loss, codebook_loss, total_quantizer_loss,
            total_tokenizer_loss)


# speedup vs baseline: 1.0596x; 1.0321x over previous
import jax
import jax.numpy as jnp
from jax.experimental import pallas as pl
from jax.experimental.pallas import tpu as pltpu

OBS_DIM = 512
ACT_DIM = 32
HID = 1024
LAT = 64
K = 1024
RB = 2048


def _body(obs_ref, act_ref, We1_ref, be1_ref, We2_ref, be2_ref, We3_ref,
          be3_ref, cb_ref, c2_ref, Wd1_ref, bd1_ref, Wd2_ref,
          bd2_ref, Wd3_ref, bd3_ref,
          recon_ref, tok_ref, qst_ref, lat_ref, vqp_ref, rcp_ref):
    f32 = jnp.float32
    x = obs_ref[...]
    a = act_ref[...]
    enc_in = jnp.concatenate((x, a), axis=1)
    h = jnp.maximum(jnp.dot(enc_in, We1_ref[...], preferred_element_type=f32)
                    + be1_ref[...], 0.0)
    h = jnp.maximum(jnp.dot(h, We2_ref[...], preferred_element_type=f32)
                    + be2_ref[...], 0.0)
    lat = jnp.dot(h, We3_ref[...], preferred_element_type=f32) + be3_ref[...]

    x2 = jnp.sum(lat * lat, axis=1, keepdims=True)
    xc = jax.lax.dot_general(lat, cb_ref[...], (((1,), (1,)), ((), ())),
                             preferred_element_type=f32)
    d2 = x2 - 2.0 * xc + c2_ref[...]
    col = jax.lax.broadcasted_iota(jnp.int32, (RB, K), 1)
    idx = jnp.argmin(d2, axis=1).astype(jnp.int32)

    onehot = (col == idx[:, None]).astype(f32)
    quant = jnp.dot(onehot, cb_ref[...], preferred_element_type=f32)
    qst = lat + (quant - lat)

    hd = jnp.maximum(
        jnp.dot(qst, Wd1_ref[...][:LAT], preferred_element_type=f32)
        + jnp.dot(a, Wd1_ref[...][LAT:], preferred_element_type=f32)
        + bd1_ref[...], 0.0)
    hd = jnp.maximum(
        jnp.dot(hd, Wd2_ref[...], preferred_element_type=f32)
        + bd2_ref[...], 0.0)
    recon = jnp.dot(hd, Wd3_ref[...], preferred_element_type=f32) + bd3_ref[...]

    recon_ref[...] = recon
    tok_ref[...] = idx[:, None]
    qst_ref[...] = qst
    lat_ref[...] = lat
    vqp_ref[...] = jnp.broadcast_to(jnp.sum((lat - quant) ** 2), (1, 1, 128))
    rcp_ref[...] = jnp.broadcast_to(jnp.sum((recon - x) ** 2), (1, 1, 128))


def kernel(obs, actions, We1, be1, We2, be2, We3, be3, codebook,
           Wd1, bd1, Wd2, bd2, Wd3, bd3):
    b, s = obs.shape[0], obs.shape[1]
    n = b * s
    grid = n // RB
    f32 = jnp.float32

    obs_f = obs.reshape(n, OBS_DIM)
    act_f = actions.reshape(n, ACT_DIM)
    c2 = jnp.sum(codebook * codebook, axis=1)[None, :]

    row = lambda rb, cols: pl.BlockSpec((rb, cols), lambda i: (i, 0))
    rep = lambda shape: pl.BlockSpec(shape, lambda i: (0,) * len(shape))

    out_shapes = (
        jax.ShapeDtypeStruct((n, OBS_DIM), f32),
        jax.ShapeDtypeStruct((n, 1), jnp.int32),
        jax.ShapeDtypeStruct((n, LAT), f32),
        jax.ShapeDtypeStruct((n, LAT), f32),
        jax.ShapeDtypeStruct((grid, 1, 128), f32),
        jax.ShapeDtypeStruct((grid, 1, 128), f32),
    )
    out_specs = (
        row(RB, OBS_DIM),
        row(RB, 1),
        row(RB, LAT),
        row(RB, LAT),
        pl.BlockSpec((1, 1, 128), lambda i: (i, 0, 0)),
        pl.BlockSpec((1, 1, 128), lambda i: (i, 0, 0)),
    )
    in_specs = (
        row(RB, OBS_DIM),
        row(RB, ACT_DIM),
        rep((OBS_DIM + ACT_DIM, HID)),
        rep((1, HID)),
        rep((HID, HID)),
        rep((1, HID)),
        rep((HID, LAT)),
        rep((1, LAT)),
        rep((K, LAT)),
        rep((1, K)),
        rep((LAT + ACT_DIM, HID)),
        rep((1, HID)),
        rep((HID, HID)),
        rep((1, HID)),
        rep((HID, OBS_DIM)),
        rep((1, OBS_DIM)),
    )

    recon_f, tok, qst_f, lat_f, vqp, rcp = pl.pallas_call(
        _body,
        grid=(grid,),
        in_specs=list(in_specs),
        out_specs=list(out_specs),
        out_shape=out_shapes,
        compiler_params=pltpu.CompilerParams(
            vmem_limit_bytes=100 * 1024 * 1024,
        ),
    )(obs_f, act_f, We1, be1[None, :], We2, be2[None, :], We3, be3[None, :],
      codebook, c2, Wd1, bd1[None, :], Wd2, bd2[None, :], Wd3, bd3[None, :])

    reconstructed_obs = recon_f.reshape(b, s, OBS_DIM)
    tokens = tok.reshape(b, s)
    quantized_st = qst_f.reshape(b, s, LAT)
    latents = lat_f.reshape(b, s, LAT)

    ssq_vq = jnp.sum(vqp[:, 0, 0])
    ssq_rc = jnp.sum(rcp[:, 0, 0])
    msd = ssq_vq / jnp.float32(n * LAT)
    commitment_loss = msd * 0.25
    codebook_loss = msd
    recon_loss = ssq_rc / jnp.float32(n * OBS_DIM)
    total_quantizer_loss = commitment_loss + codebook_loss
    total_tokenizer_loss = recon_loss + total_quantizer_loss
    return (reconstructed_obs, tokens, quantized_st, latents, recon_loss,
            commitment_loss, codebook_loss, total_quantizer_loss,
            total_tokenizer_loss)
